# baseline (speedup 1.0000x reference)
import functools

import jax
import jax.numpy as jnp
from jax import lax
from jax.experimental import pallas as pl
from jax.experimental.pallas import tpu as pltpu
from jax.experimental.pallas import tpu_sc as plsc

N_FIELDS = 26
VOCAB = 100000
DIM = 128
BATCH = 16384

_INFO = plsc.get_sparse_core_info()
_NC = _INFO.num_cores
_NS = _INFO.num_subcores
_NW = _NC * _NS
_BPW = BATCH // _NW
_LANES = 16
_ICHUNK = 128
_NCH = _BPW // _ICHUNK

_MESH = plsc.VectorSubcoreMesh(core_axis_name="c", subcore_axis_name="s")


@functools.partial(
    pl.kernel,
    out_type=jax.ShapeDtypeStruct((BATCH, DIM), jnp.float32),
    mesh=_MESH,
    scratch_types=[
        pltpu.VMEM((N_FIELDS, _NCH, _ICHUNK), jnp.int32),
        pltpu.VMEM((_BPW, DIM), jnp.float32),
        pltpu.SemaphoreType.DMA((_NCH,)),
        pltpu.SemaphoreType.DMA,
    ],
)
def _embed_sum(inputs_hbm, tables_hbm, out_hbm, idx_v, acc_v, sems, wb_sem):
    wid = lax.axis_index("s") * _NC + lax.axis_index("c")
    cbase = wid * _NCH

    pltpu.sync_copy(inputs_hbm.at[:, pl.ds(cbase, _NCH)], idx_v)

    inits = [
        pltpu.async_copy(
            tables_hbm.at[0].at[idx_v.at[0, c]],
            acc_v.at[pl.ds(c * _ICHUNK, _ICHUNK)],
            sems.at[c],
        )
        for c in range(_NCH)
    ]
    adds = []
    for c in range(_NCH):
        inits[c].wait()
        adds.append([
            pltpu.async_copy(
                tables_hbm.at[f].at[idx_v.at[f, c]],
                acc_v.at[pl.ds(c * _ICHUNK, _ICHUNK)],
                sems.at[c],
                add=True,
            )
            for f in range(1, N_FIELDS)
        ])

    wbs = []
    for c in range(_NCH):
        for d in adds[c]:
            d.wait()
        wbs.append(
            pltpu.async_copy(
                acc_v.at[pl.ds(c * _ICHUNK, _ICHUNK)],
                out_hbm.at[pl.ds((cbase + c) * _ICHUNK, _ICHUNK)],
                wb_sem,
            )
        )
    for d in wbs:
        d.wait()


def kernel(inputs, tables):
    inputs3d = inputs.reshape(N_FIELDS, BATCH // _ICHUNK, _ICHUNK)
    return _embed_sum(inputs3d, tables)


# speedup vs baseline: 1.0289x; 1.0289x over previous
import functools

import jax
import jax.numpy as jnp
from jax import lax
from jax.experimental import pallas as pl
from jax.experimental.pallas import tpu as pltpu
from jax.experimental.pallas import tpu_sc as plsc

N_FIELDS = 26
VOCAB = 100000
DIM = 128
BATCH = 16384

_INFO = plsc.get_sparse_core_info()
_NC = _INFO.num_cores
_NS = _INFO.num_subcores
_NW = _NC * _NS
_BPW = BATCH // _NW
_LANES = 16
_ICHUNK = 128
_NCH = _BPW // _ICHUNK

_MESH = plsc.VectorSubcoreMesh(core_axis_name="c", subcore_axis_name="s")


@functools.partial(
    pl.kernel,
    out_type=jax.ShapeDtypeStruct((BATCH, DIM), jnp.float32),
    mesh=_MESH,
    scratch_types=[
        pltpu.VMEM((N_FIELDS, _NCH, _ICHUNK), jnp.int32),
        pltpu.VMEM((_BPW, DIM), jnp.float32),
        pltpu.SemaphoreType.DMA((_NCH,)),
        pltpu.SemaphoreType.DMA,
    ],
)
def _embed_sum(inputs_hbm, tables_hbm, out_hbm, idx_v, acc_v, sems, wb_sem):
    wid = lax.axis_index("s") * _NC + lax.axis_index("c")
    cbase = wid * _NCH

    pltpu.sync_copy(inputs_hbm.at[:, pl.ds(cbase, _NCH)], idx_v)

    inits = [
        pltpu.async_copy(
            tables_hbm.at[idx_v.at[0, c]],
            acc_v.at[pl.ds(c * _ICHUNK, _ICHUNK)],
            sems.at[c],
        )
        for c in range(_NCH)
    ]

    def _rebase(i, carry):
        for f in range(1, N_FIELDS):
            for c in range(_NCH):
                sl = idx_v[f, c, pl.ds(i * _LANES, _LANES)]
                idx_v[f, c, pl.ds(i * _LANES, _LANES)] = sl + f * VOCAB
        return carry

    lax.fori_loop(0, _ICHUNK // _LANES, _rebase, 0)

    adds = []
    for c in range(_NCH):
        inits[c].wait()
        adds.append([
            pltpu.async_copy(
                tables_hbm.at[idx_v.at[f, c]],
                acc_v.at[pl.ds(c * _ICHUNK, _ICHUNK)],
                sems.at[c],
                add=True,
            )
            for f in range(1, N_FIELDS)
        ])

    wbs = []
    for c in range(_NCH):
        for d in adds[c]:
            d.wait()
        wbs.append(
            pltpu.async_copy(
                acc_v.at[pl.ds(c * _ICHUNK, _ICHUNK)],
                out_hbm.at[pl.ds((cbase + c) * _ICHUNK, _ICHUNK)],
                wb_sem,
            )
        )
    for d in wbs:
        d.wait()


def kernel(inputs, tables):
    flat_tables = tables.reshape(N_FIELDS * VOCAB, DIM)
    inputs3d = inputs.reshape(N_FIELDS, BATCH // _ICHUNK, _ICHUNK)
    return _embed_sum(inputs3d, flat_tables)


# speedup vs baseline: 1.0344x; 1.0053x over previous
import functools

import jax
import jax.numpy as jnp
from jax import lax
from jax.experimental import pallas as pl
from jax.experimental.pallas import tpu as pltpu
from jax.experimental.pallas import tpu_sc as plsc

N_FIELDS = 26
VOCAB = 100000
DIM = 128
BATCH = 16384

_INFO = plsc.get_sparse_core_info()
_NC = _INFO.num_cores
_NS = _INFO.num_subcores
_NW = _NC * _NS
_BPW = BATCH // _NW
_LANES = 16
_ICHUNK = 128
_NCH = _BPW // _ICHUNK

_MESH = plsc.VectorSubcoreMesh(core_axis_name="c", subcore_axis_name="s")


@functools.partial(
    pl.kernel,
    out_type=jax.ShapeDtypeStruct((BATCH, DIM), jnp.float32),
    mesh=_MESH,
    scratch_types=[
        pltpu.VMEM((N_FIELDS, _NCH, _ICHUNK), jnp.int32),
        pltpu.VMEM((_BPW, DIM), jnp.float32),
        pltpu.SemaphoreType.DMA((_NCH,)),
        pltpu.SemaphoreType.DMA,
        pltpu.SemaphoreType.DMA,
    ],
)
def _embed_sum(inputs_hbm, tables_hbm, out_hbm, idx_v, acc_v, sems, wb_sem,
               idx_sem):
    wid = lax.axis_index("s") * _NC + lax.axis_index("c")
    cbase = wid * _NCH

    ld0 = pltpu.async_copy(
        inputs_hbm.at[pl.ds(0, 1), pl.ds(cbase, _NCH)],
        idx_v.at[pl.ds(0, 1)], idx_sem)
    ld1 = pltpu.async_copy(
        inputs_hbm.at[pl.ds(1, N_FIELDS - 1), pl.ds(cbase, _NCH)],
        idx_v.at[pl.ds(1, N_FIELDS - 1)], idx_sem)
    ld0.wait()

    inits = [
        pltpu.async_copy(
            tables_hbm.at[idx_v.at[0, c]],
            acc_v.at[pl.ds(c * _ICHUNK, _ICHUNK)],
            sems.at[c],
        )
        for c in range(_NCH)
    ]

    ld1.wait()

    def _rebase(i, carry):
        for f in range(1, N_FIELDS):
            for c in range(_NCH):
                sl = idx_v[f, c, pl.ds(i * _LANES, _LANES)]
                idx_v[f, c, pl.ds(i * _LANES, _LANES)] = sl + f * VOCAB
        return carry

    lax.fori_loop(0, _ICHUNK // _LANES, _rebase, 0)

    adds = []
    for c in range(_NCH):
        inits[c].wait()
        adds.append([
            pltpu.async_copy(
                tables_hbm.at[idx_v.at[f, c]],
                acc_v.at[pl.ds(c * _ICHUNK, _ICHUNK)],
                sems.at[c],
                add=True,
            )
            for f in range(1, N_FIELDS)
        ])

    wbs = []
    for c in range(_NCH):
        for d in adds[c]:
            d.wait()
        wbs.append(
            pltpu.async_copy(
                acc_v.at[pl.ds(c * _ICHUNK, _ICHUNK)],
                out_hbm.at[pl.ds((cbase + c) * _ICHUNK, _ICHUNK)],
                wb_sem,
            )
        )
    for d in wbs:
        d.wait()


def kernel(inputs, tables):
    flat_tables = tables.reshape(N_FIELDS * VOCAB, DIM)
    inputs3d = inputs.reshape(N_FIELDS, BATCH // _ICHUNK, _ICHUNK)
    return _embed_sum(inputs3d, flat_tables)
